# baseline (speedup 1.0000x reference)
import functools

import jax
import jax.numpy as jnp
from jax import lax
from jax.experimental import pallas as pl
from jax.experimental.pallas import tpu as pltpu
from jax.experimental.pallas import tpu_sc as plsc

B, NODE, REL, NFEAT = 16, 6, 6, 64
DIM_V, DIM_WORD, DIM_VISION, DIM_EDGE, CLS_FC = 256, 512, 2048, 256, 1024
BK = B * NODE
BN = B * NFEAT
GRID = 4
Bh = B // GRID
BKh = BK // GRID
BNh = BN // GRID

_F32 = jnp.float32
_BF16 = jnp.bfloat16


def _compress(all_lr, rowb):
    acc = jnp.zeros((BKh, NFEAT), _F32)
    for j in range(Bh):
        acc = acc + jnp.where(rowb == j,
                              all_lr[:, j * NFEAT:(j + 1) * NFEAT], 0.0)
    return acc


def _cmr_body(featT_ref, node_ref, rel_ref, obj_ref, subj_ref, relm_ref,
              boxmrep_ref, boxm_ref, nodem_ref, scale_ref,
              Wmap_ref, Wedge_ref, Wnode_ref, Wrel_ref, Wfcv_ref, bfcv_ref,
              mem_out, att_out):
    X = featT_ref[...]

    x32 = X.astype(_F32)
    sq = jnp.sum(x32 * x32, axis=1, keepdims=True)
    inv = 1.0 / jnp.sqrt(sq + 1e-12)

    Xs = X * scale_ref[...]
    fmap = jnp.dot(Xs, Wmap_ref[...],
                   preferred_element_type=_F32) * inv
    fmap_b = fmap.astype(_BF16)

    We0 = Wedge_ref[0:DIM_V, :]
    We1 = Wedge_ref[DIM_V:2 * DIM_V, :]
    A0 = jnp.dot(fmap_b, We0, preferred_element_type=_F32)
    C0 = jnp.dot(fmap_b, We1, preferred_element_type=_F32)

    node_proj = jnp.dot(node_ref[...], Wnode_ref[...],
                        preferred_element_type=_F32)
    rel_proj = jnp.dot(rel_ref[...], Wrel_ref[...],
                       preferred_element_type=_F32)

    rowb = jax.lax.broadcasted_iota(jnp.int32, (BKh, 1), 0) // NODE

    L_all = jax.lax.dot_general(node_proj, fmap,
                                (((1,), (1,)), ((), ())),
                                preferred_element_type=_F32)
    logits = _compress(L_all, rowb)
    logits = jnp.where(boxmrep_ref[...] > 0.0, logits, -1e7)
    logits = logits - jnp.max(logits, axis=-1, keepdims=True)
    e = jnp.exp(logits)
    find = e / jnp.sum(e, axis=-1, keepdims=True)
    find = find * nodem_ref[...]

    P = _compress(jax.lax.dot_general(rel_proj, A0,
                                      (((1,), (1,)), ((), ())),
                                      preferred_element_type=_F32), rowb)
    Q = _compress(jax.lax.dot_general(rel_proj, C0,
                                      (((1,), (1,)), ((), ())),
                                      preferred_element_type=_F32), rowb)

    I = jax.lax.broadcasted_iota(jnp.int32, (BKh, BKh), 0)
    J = jax.lax.broadcasted_iota(jnp.int32, (BKh, BKh), 1)
    same_b = (I // NODE) == (J // NODE)
    obj = obj_ref[...]
    subj = subj_ref[0]
    OH = jnp.where(same_b & ((J % NODE) == jnp.clip(obj, 0, NODE - 1)),
                   1.0, 0.0).astype(_F32)
    SOH = jnp.where(same_b & (jnp.clip(subj, 0, NODE - 1) == (I % NODE))
                    & (subj != -1),
                    1.0, 0.0).astype(_F32)

    g = jnp.dot(OH, find, preferred_element_type=_F32)

    w = jax.nn.sigmoid(Q[:, :, None] + P[:, None, :])
    gathered = jnp.sum(g[:, :, None] * w, axis=1)
    gathered = gathered * relm_ref[...]

    find2 = find + jnp.dot(SOH, gathered, preferred_element_type=_F32)

    final_att = jnp.max(find2.reshape(Bh, NODE, NFEAT), axis=1)
    norm = jnp.maximum(jnp.max(final_att, axis=1, keepdims=True), 1.0)
    final_att = final_att / norm
    boxm = boxm_ref[0]
    final_att = final_att * boxm + (1.0 - boxm) * 1e-7
    att_out[0] = final_att

    X3 = X.reshape(Bh, NFEAT, DIM_VISION)
    mem = jnp.sum(final_att[:, :, None] * X3, axis=1)
    mem_out[0] = jnp.dot(mem.astype(_BF16), Wfcv_ref[...],
                         preferred_element_type=_F32) + bfcv_ref[...]


def _run(node_rep, relate_rep, relate_os, relate_mask, vision_feat,
         relation_mask, box_mask, node_mask, scale, W_map_v, W_edge, W_node,
         W_rel, W_fcv, b_fcv, interpret=False):
    featT = jnp.transpose(vision_feat, (0, 2, 1)).reshape(BN, DIM_VISION)
    featT = featT.astype(_BF16)
    node_flat = node_rep.reshape(BK, DIM_WORD).astype(_BF16)
    rel_flat = relate_rep.reshape(BK, DIM_WORD).astype(_BF16)
    obj_col = relate_os[:, :, 0].reshape(BK, 1)
    subj_row = relate_os[:, :, 1].reshape(GRID, 1, BKh)
    relm_flat = relate_mask.reshape(BK, 1)
    boxm_rep = jnp.repeat(box_mask, NODE, axis=0)
    nodem_flat = node_mask.reshape(BK, 1)
    scale_row = scale.reshape(1, DIM_VISION).astype(_BF16)
    bfcv2 = b_fcv.reshape(1, CLS_FC)

    def half0(shape):
        return pl.BlockSpec(shape, lambda h: (h, 0))

    def const(shape):
        return pl.BlockSpec(shape, lambda h: (0,) * len(shape))

    grid_spec = pl.GridSpec(
        grid=(GRID,),
        in_specs=[
            half0((BNh, DIM_VISION)),
            half0((BKh, DIM_WORD)),
            half0((BKh, DIM_WORD)),
            half0((BKh, 1)),
            pl.BlockSpec((1, 1, BKh), lambda h: (h, 0, 0)),
            half0((BKh, 1)),
            half0((BKh, NFEAT)),
            pl.BlockSpec((1, Bh, NFEAT), lambda h: (h, 0, 0)),
            half0((BKh, 1)),
            const((1, DIM_VISION)),
            const((DIM_VISION, DIM_V)),
            const((2 * DIM_V, DIM_EDGE)),
            const((DIM_WORD, DIM_V)),
            const((DIM_WORD, DIM_EDGE)),
            const((DIM_VISION, CLS_FC)),
            const((1, CLS_FC)),
        ],
        out_specs=[
            pl.BlockSpec((1, Bh, CLS_FC), lambda h: (h, 0, 0)),
            pl.BlockSpec((1, Bh, NFEAT), lambda h: (h, 0, 0)),
        ],
    )
    final_mem, final_att = pl.pallas_call(
        _cmr_body,
        grid_spec=grid_spec,
        out_shape=[
            jax.ShapeDtypeStruct((GRID, Bh, CLS_FC), _F32),
            jax.ShapeDtypeStruct((GRID, Bh, NFEAT), _F32),
        ],
        interpret=interpret,
    )(featT, node_flat, rel_flat, obj_col, subj_row, relm_flat, boxm_rep,
      box_mask.reshape(GRID, Bh, NFEAT), nodem_flat, scale_row,
      W_map_v.astype(_BF16), W_edge.astype(_BF16), W_node.astype(_BF16),
      W_rel.astype(_BF16), W_fcv.astype(_BF16), bfcv2)
    return final_mem.reshape(B, CLS_FC), final_att.reshape(B, NFEAT)


_KN = NODE * NFEAT


def _sc_route_body(find_hbm, p_hbm, q_hbm, obj_hbm, subj_hbm, relm_hbm,
                   boxm_hbm, att_hbm, find_v, p_v, q_v, obj_v, subj_v,
                   relm_v, boxm_v, att_v, f2_v):
    wid = lax.axis_index("s") * 2 + lax.axis_index("c")

    @pl.when(wid < B)
    def _():
        b = wid
        pltpu.sync_copy(find_hbm.at[pl.ds(b * _KN, _KN)], find_v)
        pltpu.sync_copy(p_hbm.at[pl.ds(b * _KN, _KN)], p_v)
        pltpu.sync_copy(q_hbm.at[pl.ds(b * _KN, _KN)], q_v)
        pltpu.sync_copy(obj_hbm.at[pl.ds(b * 16, 16)], obj_v)
        pltpu.sync_copy(subj_hbm.at[pl.ds(b * 16, 16)], subj_v)
        pltpu.sync_copy(relm_hbm.at[pl.ds(b * 16, 16)], relm_v)
        pltpu.sync_copy(boxm_hbm.at[pl.ds(b * NFEAT, NFEAT)], boxm_v)

        for mc in range(_KN // 16):
            f2_v[pl.ds(mc * 16, 16)] = find_v[pl.ds(mc * 16, 16)]

        objv = obj_v[...]
        subv = subj_v[...]
        relmv = relm_v[...]
        zero_f = jnp.zeros((16,), _F32)
        zero_i = jnp.zeros((16,), jnp.int32)

        def _eqf(x, c):
            return 1.0 - jnp.minimum(jnp.abs(x - c), 1.0)

        def _splat(vec, j):
            return vec.at[zero_i + j].get(mode="promise_in_bounds")

        for r in range(REL):
            obj_r = _splat(objv, r)
            sub_r = _splat(subv, r)
            relm_r = _splat(relmv, r)
            objf = jnp.clip(obj_r, 0, NODE - 1).astype(_F32)
            subf = jnp.clip(sub_r, 0, NODE - 1).astype(_F32)
            validf = 1.0 - _eqf(sub_r.astype(_F32), -1.0)

            g = []
            for mc in range(4):
                gacc = zero_f
                for k in range(NODE):
                    fk = find_v[pl.ds(k * NFEAT + mc * 16, 16)]
                    gacc = gacc + fk * _eqf(objf, float(k))
                g.append(gacc)
            p_c = [p_v[pl.ds(r * NFEAT + mc * 16, 16)] for mc in range(4)]

            acc = [zero_f, zero_f, zero_f, zero_f]
            for nd in range(4):
                qc = q_v[pl.ds(r * NFEAT + nd * 16, 16)]
                gc = g[nd]

                def nbody(nr, carry, _qc=qc, _gc=gc, _pc=p_c):
                    a0, a1, a2, a3 = carry
                    qn = _qc.at[zero_i + nr].get(mode="promise_in_bounds")
                    gn = _gc.at[zero_i + nr].get(mode="promise_in_bounds")
                    w0 = gn / (1.0 + jnp.exp(-(_pc[0] + qn)))
                    w1 = gn / (1.0 + jnp.exp(-(_pc[1] + qn)))
                    w2 = gn / (1.0 + jnp.exp(-(_pc[2] + qn)))
                    w3 = gn / (1.0 + jnp.exp(-(_pc[3] + qn)))
                    return (a0 + w0, a1 + w1, a2 + w2, a3 + w3)

                acc = list(lax.fori_loop(0, 16, nbody, tuple(acc)))

            for mc in range(4):
                a_m = acc[mc] * relm_r * validf
                for k in range(NODE):
                    off = k * NFEAT + mc * 16
                    cur = f2_v[pl.ds(off, 16)]
                    f2_v[pl.ds(off, 16)] = cur + a_m * _eqf(subf, float(k))

        for mc in range(4):
            m = f2_v[pl.ds(mc * 16, 16)]
            for k in range(1, NODE):
                m = jnp.maximum(m, f2_v[pl.ds(k * NFEAT + mc * 16, 16)])
            att_v[pl.ds(mc * 16, 16)] = m
        m01 = jnp.maximum(att_v[pl.ds(0, 16)], att_v[pl.ds(16, 16)])
        m23 = jnp.maximum(att_v[pl.ds(32, 16)], att_v[pl.ds(48, 16)])
        mall = jnp.maximum(m01, m23)
        lane = lax.broadcasted_iota(jnp.int32, (16,), 0)
        for sh in (8, 4, 2, 1):
            mall = jnp.maximum(
                mall, mall.at[lane ^ sh].get(mode="promise_in_bounds"))
        nv = jnp.maximum(mall, 1.0)
        for mc in range(4):
            bm = boxm_v[pl.ds(mc * 16, 16)]
            av = att_v[pl.ds(mc * 16, 16)] / nv
            att_v[pl.ds(mc * 16, 16)] = av * bm + (1.0 - bm) * 1e-7
        pltpu.sync_copy(att_v, att_hbm.at[pl.ds(b * NFEAT, NFEAT)])


def _sc_route(find_flat, p_flat, q_flat, obj_pad, subj_pad, relm_pad,
              boxm_flat):
    mesh = plsc.VectorSubcoreMesh(core_axis_name="c", subcore_axis_name="s")
    call = pl.kernel(
        _sc_route_body, mesh=mesh,
        out_type=jax.ShapeDtypeStruct((B * NFEAT,), _F32),
        scratch_types=[
            pltpu.VMEM((_KN,), _F32),
            pltpu.VMEM((_KN,), _F32),
            pltpu.VMEM((_KN,), _F32),
            pltpu.VMEM((16,), jnp.int32),
            pltpu.VMEM((16,), jnp.int32),
            pltpu.VMEM((16,), _F32),
            pltpu.VMEM((NFEAT,), _F32),
            pltpu.VMEM((NFEAT,), _F32),
            pltpu.VMEM((_KN,), _F32),
        ],
    )
    return call(find_flat, p_flat, q_flat, obj_pad, subj_pad, relm_pad,
                boxm_flat)


def _tc_front_body(featT_ref, node_ref, rel_ref, boxmrep_ref, nodem_ref,
                   scale_ref, Wmap_ref, Wedge_ref, Wnode_ref, Wrel_ref,
                   find_out, p_out, q_out):
    X = featT_ref[...]
    x32 = X.astype(_F32)
    sq = jnp.sum(x32 * x32, axis=1, keepdims=True)
    inv = 1.0 / jnp.sqrt(sq + 1e-12)
    Xs = X * scale_ref[...]
    fmap = jnp.dot(Xs, Wmap_ref[...],
                   preferred_element_type=_F32) * inv
    fmap_b = fmap.astype(_BF16)
    We0 = Wedge_ref[0:DIM_V, :]
    We1 = Wedge_ref[DIM_V:2 * DIM_V, :]
    A0 = jnp.dot(fmap_b, We0, preferred_element_type=_F32)
    C0 = jnp.dot(fmap_b, We1, preferred_element_type=_F32)
    node_proj = jnp.dot(node_ref[...], Wnode_ref[...],
                        preferred_element_type=_F32)
    rel_proj = jnp.dot(rel_ref[...], Wrel_ref[...],
                       preferred_element_type=_F32)
    rowb = jax.lax.broadcasted_iota(jnp.int32, (BKh, 1), 0) // NODE
    L_all = jax.lax.dot_general(node_proj, fmap, (((1,), (1,)), ((), ())),
                                preferred_element_type=_F32)
    logits = _compress(L_all, rowb)
    logits = jnp.where(boxmrep_ref[...] > 0.0, logits, -1e7)
    logits = logits - jnp.max(logits, axis=-1, keepdims=True)
    e = jnp.exp(logits)
    find = e / jnp.sum(e, axis=-1, keepdims=True)
    find_out[...] = find * nodem_ref[...]
    p_out[...] = _compress(
        jax.lax.dot_general(rel_proj, A0, (((1,), (1,)), ((), ())),
                            preferred_element_type=_F32), rowb)
    q_out[...] = _compress(
        jax.lax.dot_general(rel_proj, C0, (((1,), (1,)), ((), ())),
                            preferred_element_type=_F32), rowb)


def _tc_back_body(att_ref, featT_ref, Wfcv_ref, bfcv_ref, mem_out):
    final_att = att_ref[...]
    X3 = featT_ref[...].reshape(B, NFEAT, DIM_VISION)
    mem = jnp.sum(final_att[:, :, None] * X3, axis=1)
    mem_out[...] = jnp.dot(mem.astype(_BF16), Wfcv_ref[...],
                           preferred_element_type=_F32) + bfcv_ref[...]


def _run_sc(node_rep, relate_rep, relate_os, relate_mask, vision_feat,
            relation_mask, box_mask, node_mask, scale, W_map_v, W_edge,
            W_node, W_rel, W_fcv, b_fcv):
    featT = jnp.transpose(vision_feat, (0, 2, 1)).reshape(BN, DIM_VISION)
    featT = featT.astype(_BF16)
    node_flat = node_rep.reshape(BK, DIM_WORD).astype(_BF16)
    rel_flat = relate_rep.reshape(BK, DIM_WORD).astype(_BF16)
    boxm_rep = jnp.repeat(box_mask, NODE, axis=0)
    nodem_flat = node_mask.reshape(BK, 1)
    scale_row = scale.reshape(1, DIM_VISION).astype(_BF16)
    bfcv2 = b_fcv.reshape(1, CLS_FC)

    def half0(shape):
        return pl.BlockSpec(shape, lambda h: (h, 0))

    def const(shape):
        return pl.BlockSpec(shape, lambda h: (0,) * len(shape))

    grid_spec = pl.GridSpec(
        grid=(GRID,),
        in_specs=[
            half0((BNh, DIM_VISION)),
            half0((BKh, DIM_WORD)),
            half0((BKh, DIM_WORD)),
            half0((BKh, NFEAT)),
            half0((BKh, 1)),
            const((1, DIM_VISION)),
            const((DIM_VISION, DIM_V)),
            const((2 * DIM_V, DIM_EDGE)),
            const((DIM_WORD, DIM_V)),
            const((DIM_WORD, DIM_EDGE)),
        ],
        out_specs=[half0((BKh, NFEAT))] * 3,
    )
    find, Pm, Qm = pl.pallas_call(
        _tc_front_body,
        grid_spec=grid_spec,
        out_shape=[jax.ShapeDtypeStruct((BK, NFEAT), _F32)] * 3,
    )(featT, node_flat, rel_flat, boxm_rep, nodem_flat, scale_row,
      W_map_v.astype(_BF16), W_edge.astype(_BF16), W_node.astype(_BF16),
      W_rel.astype(_BF16))

    obj_pad = jnp.pad(relate_os[:, :, 0], ((0, 0), (0, 16 - REL))).reshape(-1)
    subj_pad = jnp.pad(relate_os[:, :, 1], ((0, 0), (0, 16 - REL))).reshape(-1)
    relm_pad = jnp.pad(relate_mask, ((0, 0), (0, 16 - REL))).reshape(-1)
    att_flat = _sc_route(find.reshape(-1), Pm.reshape(-1), Qm.reshape(-1),
                         obj_pad, subj_pad, relm_pad, box_mask.reshape(-1))
    final_att = att_flat.reshape(B, NFEAT)

    final_mem = pl.pallas_call(
        _tc_back_body,
        out_shape=jax.ShapeDtypeStruct((B, CLS_FC), _F32),
    )(final_att, featT, W_fcv.astype(_BF16), bfcv2)
    return final_mem, final_att


def kernel(node_rep, relate_rep, relate_os, relate_mask, vision_feat,
           relation_mask, box_mask, node_mask, scale, W_map_v, W_edge,
           W_node, W_rel, W_fcv, b_fcv):
    return _run(node_rep, relate_rep, relate_os, relate_mask, vision_feat,
                relation_mask, box_mask, node_mask, scale, W_map_v, W_edge,
                W_node, W_rel, W_fcv, b_fcv)


# speedup vs baseline: 1.0769x; 1.0769x over previous
import functools

import jax
import jax.numpy as jnp
from jax import lax
from jax.experimental import pallas as pl
from jax.experimental.pallas import tpu as pltpu
from jax.experimental.pallas import tpu_sc as plsc

B, NODE, REL, NFEAT = 16, 6, 6, 64
DIM_V, DIM_WORD, DIM_VISION, DIM_EDGE, CLS_FC = 256, 512, 2048, 256, 1024
BK = B * NODE
BN = B * NFEAT
GRID = 2
Bh = B // GRID
BKh = BK // GRID
BNh = BN // GRID

_F32 = jnp.float32
_BF16 = jnp.bfloat16


def _compress(all_lr, rowb):
    acc = jnp.zeros((BKh, NFEAT), _F32)
    for j in range(Bh):
        acc = acc + jnp.where(rowb == j,
                              all_lr[:, j * NFEAT:(j + 1) * NFEAT], 0.0)
    return acc


def _cmr_body(featT_ref, node_ref, rel_ref, obj_ref, subj_ref, relm_ref,
              boxmrep_ref, boxm_ref, nodem_ref, scale_ref,
              Wmap_ref, Wedge_ref, Wnode_ref, Wrel_ref, Wfcv_ref, bfcv_ref,
              mem_out, att_out):
    X = featT_ref[...]

    x32 = X.astype(_F32)
    sq = jnp.sum(x32 * x32, axis=1, keepdims=True)
    inv = 1.0 / jnp.sqrt(sq + 1e-12)

    Xs = X * scale_ref[...]
    fmap = jnp.dot(Xs, Wmap_ref[...],
                   preferred_element_type=_F32) * inv
    fmap_b = fmap.astype(_BF16)

    We0 = Wedge_ref[0:DIM_V, :]
    We1 = Wedge_ref[DIM_V:2 * DIM_V, :]
    A0 = jnp.dot(fmap_b, We0, preferred_element_type=_F32)
    C0 = jnp.dot(fmap_b, We1, preferred_element_type=_F32)

    node_proj = jnp.dot(node_ref[...], Wnode_ref[...],
                        preferred_element_type=_F32)
    rel_proj = jnp.dot(rel_ref[...], Wrel_ref[...],
                       preferred_element_type=_F32)

    rowb = jax.lax.broadcasted_iota(jnp.int32, (BKh, 1), 0) // NODE

    L_all = jax.lax.dot_general(node_proj, fmap,
                                (((1,), (1,)), ((), ())),
                                preferred_element_type=_F32)
    logits = _compress(L_all, rowb)
    logits = jnp.where(boxmrep_ref[...] > 0.0, logits, -1e7)
    logits = logits - jnp.max(logits, axis=-1, keepdims=True)
    e = jnp.exp(logits)
    find = e / jnp.sum(e, axis=-1, keepdims=True)
    find = find * nodem_ref[...]

    P = _compress(jax.lax.dot_general(rel_proj, A0,
                                      (((1,), (1,)), ((), ())),
                                      preferred_element_type=_F32), rowb)
    Q = _compress(jax.lax.dot_general(rel_proj, C0,
                                      (((1,), (1,)), ((), ())),
                                      preferred_element_type=_F32), rowb)

    I = jax.lax.broadcasted_iota(jnp.int32, (BKh, BKh), 0)
    J = jax.lax.broadcasted_iota(jnp.int32, (BKh, BKh), 1)
    same_b = (I // NODE) == (J // NODE)
    obj = obj_ref[...]
    subj = subj_ref[0]
    OH = jnp.where(same_b & ((J % NODE) == jnp.clip(obj, 0, NODE - 1)),
                   1.0, 0.0).astype(_F32)
    SOH = jnp.where(same_b & (jnp.clip(subj, 0, NODE - 1) == (I % NODE))
                    & (subj != -1),
                    1.0, 0.0).astype(_F32)

    g = jnp.dot(OH, find, preferred_element_type=_F32)

    w = jax.nn.sigmoid(Q[:, :, None] + P[:, None, :])
    gathered = jnp.sum(g[:, :, None] * w, axis=1)
    gathered = gathered * relm_ref[...]

    find2 = find + jnp.dot(SOH, gathered, preferred_element_type=_F32)

    final_att = jnp.max(find2.reshape(Bh, NODE, NFEAT), axis=1)
    norm = jnp.maximum(jnp.max(final_att, axis=1, keepdims=True), 1.0)
    final_att = final_att / norm
    boxm = boxm_ref[0]
    final_att = final_att * boxm + (1.0 - boxm) * 1e-7
    att_out[0] = final_att

    X3 = X.reshape(Bh, NFEAT, DIM_VISION)
    mem = jnp.sum(final_att[:, :, None] * X3, axis=1)
    mem_out[0] = jnp.dot(mem.astype(_BF16), Wfcv_ref[...],
                         preferred_element_type=_F32) + bfcv_ref[...]


def _run(node_rep, relate_rep, relate_os, relate_mask, vision_feat,
         relation_mask, box_mask, node_mask, scale, W_map_v, W_edge, W_node,
         W_rel, W_fcv, b_fcv, interpret=False):
    featT = jnp.transpose(vision_feat, (0, 2, 1)).reshape(BN, DIM_VISION)
    featT = featT.astype(_BF16)
    node_flat = node_rep.reshape(BK, DIM_WORD).astype(_BF16)
    rel_flat = relate_rep.reshape(BK, DIM_WORD).astype(_BF16)
    obj_col = relate_os[:, :, 0].reshape(BK, 1)
    subj_row = relate_os[:, :, 1].reshape(GRID, 1, BKh)
    relm_flat = relate_mask.reshape(BK, 1)
    boxm_rep = jnp.repeat(box_mask, NODE, axis=0)
    nodem_flat = node_mask.reshape(BK, 1)
    scale_row = scale.reshape(1, DIM_VISION).astype(_BF16)
    bfcv2 = b_fcv.reshape(1, CLS_FC)

    def half0(shape):
        return pl.BlockSpec(shape, lambda h: (h, 0))

    def const(shape):
        return pl.BlockSpec(shape, lambda h: (0,) * len(shape))

    grid_spec = pl.GridSpec(
        grid=(GRID,),
        in_specs=[
            half0((BNh, DIM_VISION)),
            half0((BKh, DIM_WORD)),
            half0((BKh, DIM_WORD)),
            half0((BKh, 1)),
            pl.BlockSpec((1, 1, BKh), lambda h: (h, 0, 0)),
            half0((BKh, 1)),
            half0((BKh, NFEAT)),
            pl.BlockSpec((1, Bh, NFEAT), lambda h: (h, 0, 0)),
            half0((BKh, 1)),
            const((1, DIM_VISION)),
            const((DIM_VISION, DIM_V)),
            const((2 * DIM_V, DIM_EDGE)),
            const((DIM_WORD, DIM_V)),
            const((DIM_WORD, DIM_EDGE)),
            const((DIM_VISION, CLS_FC)),
            const((1, CLS_FC)),
        ],
        out_specs=[
            pl.BlockSpec((1, Bh, CLS_FC), lambda h: (h, 0, 0)),
            pl.BlockSpec((1, Bh, NFEAT), lambda h: (h, 0, 0)),
        ],
    )
    final_mem, final_att = pl.pallas_call(
        _cmr_body,
        grid_spec=grid_spec,
        out_shape=[
            jax.ShapeDtypeStruct((GRID, Bh, CLS_FC), _F32),
            jax.ShapeDtypeStruct((GRID, Bh, NFEAT), _F32),
        ],
        interpret=interpret,
    )(featT, node_flat, rel_flat, obj_col, subj_row, relm_flat, boxm_rep,
      box_mask.reshape(GRID, Bh, NFEAT), nodem_flat, scale_row,
      W_map_v.astype(_BF16), W_edge.astype(_BF16), W_node.astype(_BF16),
      W_rel.astype(_BF16), W_fcv.astype(_BF16), bfcv2)
    return final_mem.reshape(B, CLS_FC), final_att.reshape(B, NFEAT)


_KN = NODE * NFEAT


def _sc_route_body(find_hbm, p_hbm, q_hbm, obj_hbm, subj_hbm, relm_hbm,
                   boxm_hbm, att_hbm, find_v, p_v, q_v, obj_v, subj_v,
                   relm_v, boxm_v, att_v, f2_v):
    wid = lax.axis_index("s") * 2 + lax.axis_index("c")

    @pl.when(wid < B)
    def _():
        b = wid
        pltpu.sync_copy(find_hbm.at[pl.ds(b * _KN, _KN)], find_v)
        pltpu.sync_copy(p_hbm.at[pl.ds(b * _KN, _KN)], p_v)
        pltpu.sync_copy(q_hbm.at[pl.ds(b * _KN, _KN)], q_v)
        pltpu.sync_copy(obj_hbm.at[pl.ds(b * 16, 16)], obj_v)
        pltpu.sync_copy(subj_hbm.at[pl.ds(b * 16, 16)], subj_v)
        pltpu.sync_copy(relm_hbm.at[pl.ds(b * 16, 16)], relm_v)
        pltpu.sync_copy(boxm_hbm.at[pl.ds(b * NFEAT, NFEAT)], boxm_v)

        for mc in range(_KN // 16):
            f2_v[pl.ds(mc * 16, 16)] = find_v[pl.ds(mc * 16, 16)]

        objv = obj_v[...]
        subv = subj_v[...]
        relmv = relm_v[...]
        zero_f = jnp.zeros((16,), _F32)
        zero_i = jnp.zeros((16,), jnp.int32)

        def _eqf(x, c):
            return 1.0 - jnp.minimum(jnp.abs(x - c), 1.0)

        def _splat(vec, j):
            return vec.at[zero_i + j].get(mode="promise_in_bounds")

        for r in range(REL):
            obj_r = _splat(objv, r)
            sub_r = _splat(subv, r)
            relm_r = _splat(relmv, r)
            objf = jnp.clip(obj_r, 0, NODE - 1).astype(_F32)
            subf = jnp.clip(sub_r, 0, NODE - 1).astype(_F32)
            validf = 1.0 - _eqf(sub_r.astype(_F32), -1.0)

            g = []
            for mc in range(4):
                gacc = zero_f
                for k in range(NODE):
                    fk = find_v[pl.ds(k * NFEAT + mc * 16, 16)]
                    gacc = gacc + fk * _eqf(objf, float(k))
                g.append(gacc)
            p_c = [p_v[pl.ds(r * NFEAT + mc * 16, 16)] for mc in range(4)]

            acc = [zero_f, zero_f, zero_f, zero_f]
            for nd in range(4):
                qc = q_v[pl.ds(r * NFEAT + nd * 16, 16)]
                gc = g[nd]

                def nbody(nr, carry, _qc=qc, _gc=gc, _pc=p_c):
                    a0, a1, a2, a3 = carry
                    qn = _qc.at[zero_i + nr].get(mode="promise_in_bounds")
                    gn = _gc.at[zero_i + nr].get(mode="promise_in_bounds")
                    w0 = gn / (1.0 + jnp.exp(-(_pc[0] + qn)))
                    w1 = gn / (1.0 + jnp.exp(-(_pc[1] + qn)))
                    w2 = gn / (1.0 + jnp.exp(-(_pc[2] + qn)))
                    w3 = gn / (1.0 + jnp.exp(-(_pc[3] + qn)))
                    return (a0 + w0, a1 + w1, a2 + w2, a3 + w3)

                acc = list(lax.fori_loop(0, 16, nbody, tuple(acc)))

            for mc in range(4):
                a_m = acc[mc] * relm_r * validf
                for k in range(NODE):
                    off = k * NFEAT + mc * 16
                    cur = f2_v[pl.ds(off, 16)]
                    f2_v[pl.ds(off, 16)] = cur + a_m * _eqf(subf, float(k))

        for mc in range(4):
            m = f2_v[pl.ds(mc * 16, 16)]
            for k in range(1, NODE):
                m = jnp.maximum(m, f2_v[pl.ds(k * NFEAT + mc * 16, 16)])
            att_v[pl.ds(mc * 16, 16)] = m
        m01 = jnp.maximum(att_v[pl.ds(0, 16)], att_v[pl.ds(16, 16)])
        m23 = jnp.maximum(att_v[pl.ds(32, 16)], att_v[pl.ds(48, 16)])
        mall = jnp.maximum(m01, m23)
        lane = lax.broadcasted_iota(jnp.int32, (16,), 0)
        for sh in (8, 4, 2, 1):
            mall = jnp.maximum(
                mall, mall.at[lane ^ sh].get(mode="promise_in_bounds"))
        nv = jnp.maximum(mall, 1.0)
        for mc in range(4):
            bm = boxm_v[pl.ds(mc * 16, 16)]
            av = att_v[pl.ds(mc * 16, 16)] / nv
            att_v[pl.ds(mc * 16, 16)] = av * bm + (1.0 - bm) * 1e-7
        pltpu.sync_copy(att_v, att_hbm.at[pl.ds(b * NFEAT, NFEAT)])


def _sc_route(find_flat, p_flat, q_flat, obj_pad, subj_pad, relm_pad,
              boxm_flat):
    mesh = plsc.VectorSubcoreMesh(core_axis_name="c", subcore_axis_name="s")
    call = pl.kernel(
        _sc_route_body, mesh=mesh,
        out_type=jax.ShapeDtypeStruct((B * NFEAT,), _F32),
        scratch_types=[
            pltpu.VMEM((_KN,), _F32),
            pltpu.VMEM((_KN,), _F32),
            pltpu.VMEM((_KN,), _F32),
            pltpu.VMEM((16,), jnp.int32),
            pltpu.VMEM((16,), jnp.int32),
            pltpu.VMEM((16,), _F32),
            pltpu.VMEM((NFEAT,), _F32),
            pltpu.VMEM((NFEAT,), _F32),
            pltpu.VMEM((_KN,), _F32),
        ],
    )
    return call(find_flat, p_flat, q_flat, obj_pad, subj_pad, relm_pad,
                boxm_flat)


def _tc_front_body(featT_ref, node_ref, rel_ref, boxmrep_ref, nodem_ref,
                   scale_ref, Wmap_ref, Wedge_ref, Wnode_ref, Wrel_ref,
                   find_out, p_out, q_out):
    X = featT_ref[...]
    x32 = X.astype(_F32)
    sq = jnp.sum(x32 * x32, axis=1, keepdims=True)
    inv = 1.0 / jnp.sqrt(sq + 1e-12)
    Xs = X * scale_ref[...]
    fmap = jnp.dot(Xs, Wmap_ref[...],
                   preferred_element_type=_F32) * inv
    fmap_b = fmap.astype(_BF16)
    We0 = Wedge_ref[0:DIM_V, :]
    We1 = Wedge_ref[DIM_V:2 * DIM_V, :]
    A0 = jnp.dot(fmap_b, We0, preferred_element_type=_F32)
    C0 = jnp.dot(fmap_b, We1, preferred_element_type=_F32)
    node_proj = jnp.dot(node_ref[...], Wnode_ref[...],
                        preferred_element_type=_F32)
    rel_proj = jnp.dot(rel_ref[...], Wrel_ref[...],
                       preferred_element_type=_F32)
    rowb = jax.lax.broadcasted_iota(jnp.int32, (BKh, 1), 0) // NODE
    L_all = jax.lax.dot_general(node_proj, fmap, (((1,), (1,)), ((), ())),
                                preferred_element_type=_F32)
    logits = _compress(L_all, rowb)
    logits = jnp.where(boxmrep_ref[...] > 0.0, logits, -1e7)
    logits = logits - jnp.max(logits, axis=-1, keepdims=True)
    e = jnp.exp(logits)
    find = e / jnp.sum(e, axis=-1, keepdims=True)
    find_out[...] = find * nodem_ref[...]
    p_out[...] = _compress(
        jax.lax.dot_general(rel_proj, A0, (((1,), (1,)), ((), ())),
                            preferred_element_type=_F32), rowb)
    q_out[...] = _compress(
        jax.lax.dot_general(rel_proj, C0, (((1,), (1,)), ((), ())),
                            preferred_element_type=_F32), rowb)


def _tc_back_body(att_ref, featT_ref, Wfcv_ref, bfcv_ref, mem_out):
    final_att = att_ref[...]
    X3 = featT_ref[...].reshape(B, NFEAT, DIM_VISION)
    mem = jnp.sum(final_att[:, :, None] * X3, axis=1)
    mem_out[...] = jnp.dot(mem.astype(_BF16), Wfcv_ref[...],
                           preferred_element_type=_F32) + bfcv_ref[...]


def _run_sc(node_rep, relate_rep, relate_os, relate_mask, vision_feat,
            relation_mask, box_mask, node_mask, scale, W_map_v, W_edge,
            W_node, W_rel, W_fcv, b_fcv):
    featT = jnp.transpose(vision_feat, (0, 2, 1)).reshape(BN, DIM_VISION)
    featT = featT.astype(_BF16)
    node_flat = node_rep.reshape(BK, DIM_WORD).astype(_BF16)
    rel_flat = relate_rep.reshape(BK, DIM_WORD).astype(_BF16)
    boxm_rep = jnp.repeat(box_mask, NODE, axis=0)
    nodem_flat = node_mask.reshape(BK, 1)
    scale_row = scale.reshape(1, DIM_VISION).astype(_BF16)
    bfcv2 = b_fcv.reshape(1, CLS_FC)

    def half0(shape):
        return pl.BlockSpec(shape, lambda h: (h, 0))

    def const(shape):
        return pl.BlockSpec(shape, lambda h: (0,) * len(shape))

    grid_spec = pl.GridSpec(
        grid=(GRID,),
        in_specs=[
            half0((BNh, DIM_VISION)),
            half0((BKh, DIM_WORD)),
            half0((BKh, DIM_WORD)),
            half0((BKh, NFEAT)),
            half0((BKh, 1)),
            const((1, DIM_VISION)),
            const((DIM_VISION, DIM_V)),
            const((2 * DIM_V, DIM_EDGE)),
            const((DIM_WORD, DIM_V)),
            const((DIM_WORD, DIM_EDGE)),
        ],
        out_specs=[half0((BKh, NFEAT))] * 3,
    )
    find, Pm, Qm = pl.pallas_call(
        _tc_front_body,
        grid_spec=grid_spec,
        out_shape=[jax.ShapeDtypeStruct((BK, NFEAT), _F32)] * 3,
    )(featT, node_flat, rel_flat, boxm_rep, nodem_flat, scale_row,
      W_map_v.astype(_BF16), W_edge.astype(_BF16), W_node.astype(_BF16),
      W_rel.astype(_BF16))

    obj_pad = jnp.pad(relate_os[:, :, 0], ((0, 0), (0, 16 - REL))).reshape(-1)
    subj_pad = jnp.pad(relate_os[:, :, 1], ((0, 0), (0, 16 - REL))).reshape(-1)
    relm_pad = jnp.pad(relate_mask, ((0, 0), (0, 16 - REL))).reshape(-1)
    att_flat = _sc_route(find.reshape(-1), Pm.reshape(-1), Qm.reshape(-1),
                         obj_pad, subj_pad, relm_pad, box_mask.reshape(-1))
    final_att = att_flat.reshape(B, NFEAT)

    final_mem = pl.pallas_call(
        _tc_back_body,
        out_shape=jax.ShapeDtypeStruct((B, CLS_FC), _F32),
    )(final_att, featT, W_fcv.astype(_BF16), bfcv2)
    return final_mem, final_att


def kernel(node_rep, relate_rep, relate_os, relate_mask, vision_feat,
           relation_mask, box_mask, node_mask, scale, W_map_v, W_edge,
           W_node, W_rel, W_fcv, b_fcv):
    return _run(node_rep, relate_rep, relate_os, relate_mask, vision_feat,
                relation_mask, box_mask, node_mask, scale, W_map_v, W_edge,
                W_node, W_rel, W_fcv, b_fcv)
